# baseline (speedup 1.0000x reference)
import jax
import jax.numpy as jnp
from jax import lax
from jax.experimental import pallas as pl
from jax.experimental.pallas import tpu as pltpu
from jax.experimental.pallas import tpu_sc as plsc

_NC = 2
_NS = 16
_L = 16

_N = 10000
_N_PAD = 10240
_RPT = _N_PAD // _NS
_W = 128


def _seg_sum_sc(dh: int, rows_pt: int, with_count: bool):
  out_type = [jax.ShapeDtypeStruct((_NC, _N_PAD, dh), jnp.float32)]
  if with_count:
    out_type.append(jax.ShapeDtypeStruct((_NC, _N_PAD), jnp.float32))

  nbuf = 4
  nslot = 2 * nbuf
  scratch = [
      pltpu.VMEM((nslot, _W), jnp.int32),
      pltpu.VMEM((nslot, _W), jnp.int32),
  ] + [pltpu.VMEM((_W, dh), jnp.float32) for _ in range(nbuf)] + [
      pltpu.VMEM((_W,), jnp.float32),
      pltpu.VMEM((16, dh), jnp.float32),
      pltpu.VMEM((_RPT,), jnp.float32),
      pltpu.VMEM_SHARED((_N_PAD, dh), jnp.float32),
      pltpu.VMEM_SHARED((_N_PAD, dh), jnp.float32),
      pltpu.VMEM_SHARED((_N_PAD,), jnp.float32),
      pltpu.SemaphoreType.DMA((nslot,)),
      pltpu.SemaphoreType.DMA((nbuf,)),
      pltpu.SemaphoreType.DMA((nbuf,)),
      pltpu.SemaphoreType.DMA((nbuf,)),
  ]

  mesh = plsc.VectorSubcoreMesh(core_axis_name="c", subcore_axis_name="s")

  def body(y_hbm, src_hbm, dst_hbm, *out_and_scratch):
    if with_count:
      out_hbm, cnt_hbm = out_and_scratch[:2]
      sc = out_and_scratch[2:]
    else:
      out_hbm = out_and_scratch[0]
      sc = out_and_scratch[1:]
    src_v, dst_v = sc[0], sc[1]
    rows = sc[2:2 + nbuf]
    (ones_v, zbuf, zcnt, y_s, acc_s, cnt_s,
     isem, gsem, ssem, csem) = sc[2 + nbuf:]

    c = lax.axis_index("c")
    s = lax.axis_index("s")

    pltpu.sync_copy(y_hbm.at[c, pl.ds(s * _RPT, _RPT)],
                    y_s.at[pl.ds(s * _RPT, _RPT)])

    zero = jnp.zeros((_L,), jnp.float32)
    one = jnp.ones((_L,), jnp.float32)
    for i in range(16):
      for j in range(dh // _L):
        zbuf[i, pl.ds(j * _L, _L)] = zero
    for i in range(_RPT // _L):
      zcnt[pl.ds(i * _L, _L)] = zero
    for i in range(_W // _L):
      ones_v[pl.ds(i * _L, _L)] = one

    for i in range(_RPT // 16):
      pltpu.sync_copy(zbuf, acc_s.at[pl.ds(s * _RPT + i * 16, 16)])
    if with_count:
      pltpu.sync_copy(zcnt, cnt_s.at[pl.ds(s * _RPT, _RPT)])

    plsc.subcore_barrier()

    base = s * rows_pt

    def i_start(j):
      sl = j % nslot
      pltpu.async_copy(src_hbm.at[base + j], src_v.at[sl], isem.at[sl])
      pltpu.async_copy(dst_hbm.at[base + j], dst_v.at[sl], isem.at[sl])

    def i_wait(j):
      sl = j % nslot
      pltpu.make_async_copy(src_hbm.at[base + j], src_v.at[sl],
                            isem.at[sl]).wait()
      pltpu.make_async_copy(dst_hbm.at[base + j], dst_v.at[sl],
                            isem.at[sl]).wait()

    def g_start(j, b):
      pltpu.async_copy(y_s.at[src_v.at[j % nslot]], rows[b], gsem.at[b])

    def g_wait(j, b):
      pltpu.make_async_copy(y_s.at[src_v.at[j % nslot]], rows[b],
                            gsem.at[b]).wait()

    def s_start(j, b):
      pltpu.async_copy(rows[b], acc_s.at[dst_v.at[j % nslot]], ssem.at[b],
                       add=True)
      if with_count:
        pltpu.async_copy(ones_v, cnt_s.at[dst_v.at[j % nslot]], csem.at[b],
                         add=True)

    def s_wait(j, b):
      pltpu.make_async_copy(rows[b], acc_s.at[dst_v.at[j % nslot]],
                            ssem.at[b]).wait()
      if with_count:
        pltpu.make_async_copy(ones_v, cnt_s.at[dst_v.at[j % nslot]],
                              csem.at[b]).wait()

    n_it = rows_pt // nbuf
    for j in range(nslot):
      i_start(j)
    for b in range(nbuf):
      i_wait(b)
      g_start(b, b)

    def step(k, carry):
      j0 = k * nbuf
      for b in range(nbuf):
        g_wait(j0 + b, b)
        s_start(j0 + b, b)
      for b in range(nbuf):
        s_wait(j0 + b, b)
        i_wait(j0 + nbuf + b)
        g_start(j0 + nbuf + b, b)
        i_start(j0 + 2 * nbuf + b)
      return carry

    lax.fori_loop(0, n_it - 2, step, 0)

    for k in (n_it - 2, n_it - 1):
      j0 = k * nbuf
      for b in range(nbuf):
        g_wait(j0 + b, b)
        s_start(j0 + b, b)
      for b in range(nbuf):
        s_wait(j0 + b, b)
        if j0 + nbuf + b < rows_pt:
          i_wait(j0 + nbuf + b)
          g_start(j0 + nbuf + b, b)

    plsc.subcore_barrier()

    pltpu.sync_copy(acc_s.at[pl.ds(s * _RPT, _RPT)],
                    out_hbm.at[c, pl.ds(s * _RPT, _RPT)])
    if with_count:
      pltpu.sync_copy(cnt_s.at[pl.ds(s * _RPT, _RPT)],
                      cnt_hbm.at[c, pl.ds(s * _RPT, _RPT)])

  return pl.kernel(body, out_type=tuple(out_type), mesh=mesh,
                   scratch_types=scratch,
                   compiler_params=pltpu.CompilerParams(
                       use_tc_tiling_on_sc=False))


_BLK = 400
_NBLK = _N // _BLK


def _tc1_body(x_ref, wl_ref, wr_ref, y_ref, r_ref):
  xb = x_ref[...]
  dh = y_ref.shape[2]
  yb = jnp.dot(xb, wl_ref[...], preferred_element_type=jnp.float32)
  y_ref[0] = yb[:, :dh]
  y_ref[1] = yb[:, dh:]
  r_ref[...] = jnp.dot(xb, wr_ref[...], preferred_element_type=jnp.float32)


def _tc2_body(p_ref, cnt_ref, r1_ref, b1_ref, w2l_ref, w2r_ref, y2_ref,
              r2_ref):
  qh = y2_ref.shape[2]
  mean = jnp.concatenate([p_ref[0], p_ref[1]], axis=1) / cnt_ref[...]
  h = jnp.maximum(mean + b1_ref[...] + r1_ref[...], 0.0)
  y2 = jnp.dot(h, w2l_ref[...], preferred_element_type=jnp.float32)
  y2_ref[0] = y2[:, :qh]
  y2_ref[1] = y2[:, qh:]
  r2_ref[...] = jnp.dot(h, w2r_ref[...], preferred_element_type=jnp.float32)


def _tc3_body(p_ref, cnt_ref, r2_ref, b2_ref, wlin_ref, blin_ref,
              out_ref):
  mean = jnp.concatenate([p_ref[0], p_ref[1]], axis=1) / cnt_ref[...]
  g = jnp.maximum(mean + b2_ref[...] + r2_ref[...], 0.0)
  logits = (jnp.dot(g, wlin_ref[...], preferred_element_type=jnp.float32)
            + blin_ref[...])
  m = jnp.max(logits, axis=1, keepdims=True)
  lse = m + jnp.log(jnp.sum(jnp.exp(logits - m), axis=1, keepdims=True))
  out_ref[...] = logits - lse


def _full(shape):
  return pl.BlockSpec(shape, lambda i: tuple(0 for _ in shape))


def kernel(x, edge_index, W1_l, b1, W1_r, W2_l, b2, W2_r, W_lin, b_lin):
  n, d_in = x.shape
  h1 = W1_l.shape[1]
  h2 = W2_l.shape[1]
  ncls = W_lin.shape[1]
  e = edge_index.shape[1]
  dh = h1 // 2

  src = edge_index[0].astype(jnp.int32)
  dst = edge_index[1].astype(jnp.int32)

  e_pt = -(-e // _NS)
  rows_pt = -(-(-(-e_pt // _W)) // 8) * 8
  e_pad = _NS * rows_pt * _W
  pad = e_pad - e
  src2d = jnp.concatenate([src, jnp.zeros((pad,), jnp.int32)]).reshape(-1, _W)
  dst2d = jnp.concatenate(
      [dst, _N + (jnp.arange(pad, dtype=jnp.int32) % (_N_PAD - _N))]
  ).reshape(-1, _W)

  y1, r1 = pl.pallas_call(
      _tc1_body,
      grid=(_NBLK,),
      in_specs=[pl.BlockSpec((_BLK, d_in), lambda i: (i, 0)),
                _full((d_in, h1)), _full((d_in, h1))],
      out_specs=[pl.BlockSpec((_NC, _BLK, dh), lambda i: (0, i, 0)),
                 pl.BlockSpec((_BLK, h1), lambda i: (i, 0))],
      out_shape=[jax.ShapeDtypeStruct((_NC, _N_PAD, dh), jnp.float32),
                 jax.ShapeDtypeStruct((_N_PAD, h1), jnp.float32)],
  )(x, W1_l, W1_r)

  p1, c1 = _seg_sum_sc(dh, rows_pt, True)(y1, src2d, dst2d)
  cnt = jnp.maximum(c1[0], 1.0).reshape(_N_PAD, 1)

  qh = h2 // 2
  y2, r2 = pl.pallas_call(
      _tc2_body,
      grid=(_NBLK,),
      in_specs=[pl.BlockSpec((_NC, _BLK, dh), lambda i: (0, i, 0)),
                pl.BlockSpec((_BLK, 1), lambda i: (i, 0)),
                pl.BlockSpec((_BLK, h1), lambda i: (i, 0)),
                _full((1, h1)), _full((h1, h2)), _full((h1, h2))],
      out_specs=[pl.BlockSpec((_NC, _BLK, qh), lambda i: (0, i, 0)),
                 pl.BlockSpec((_BLK, h2), lambda i: (i, 0))],
      out_shape=[jax.ShapeDtypeStruct((_NC, _N_PAD, qh), jnp.float32),
                 jax.ShapeDtypeStruct((_N_PAD, h2), jnp.float32)],
  )(p1, cnt, r1, b1.reshape(1, h1), W2_l, W2_r)

  (p2,) = _seg_sum_sc(qh, rows_pt, False)(y2, src2d, dst2d)

  out = pl.pallas_call(
      _tc3_body,
      grid=(_NBLK,),
      in_specs=[pl.BlockSpec((_NC, _BLK, qh), lambda i: (0, i, 0)),
                pl.BlockSpec((_BLK, 1), lambda i: (i, 0)),
                pl.BlockSpec((_BLK, h2), lambda i: (i, 0)),
                _full((1, h2)),
                _full((h2, ncls)), _full((1, ncls))],
      out_specs=pl.BlockSpec((_BLK, ncls), lambda i: (i, 0)),
      out_shape=jax.ShapeDtypeStruct((n, ncls), jnp.float32),
  )(p2, cnt, r2, b2.reshape(1, h2), W_lin, b_lin.reshape(1, ncls))

  return out


# speedup vs baseline: 1.0406x; 1.0406x over previous
import jax
import jax.numpy as jnp
from jax import lax
from jax.experimental import pallas as pl
from jax.experimental.pallas import tpu as pltpu
from jax.experimental.pallas import tpu_sc as plsc

_NC = 2
_NS = 16
_L = 16

_N = 10000
_N_PAD = 10240
_RPT = _N_PAD // _NS
_W = 128


def _seg_sum_sc(dh: int, rows_pt: int, with_count: bool):
  out_type = [jax.ShapeDtypeStruct((_NC, _N_PAD, dh), jnp.float32)]
  if with_count:
    out_type.append(jax.ShapeDtypeStruct((_NC, _N_PAD), jnp.float32))

  nbuf = 5
  nslot = 2 * nbuf
  scratch = [
      pltpu.VMEM((nslot, _W), jnp.int32),
      pltpu.VMEM((nslot, _W), jnp.int32),
  ] + [pltpu.VMEM((_W, dh), jnp.float32) for _ in range(nbuf)] + [
      pltpu.VMEM((_W,), jnp.float32),
      pltpu.VMEM((16, dh), jnp.float32),
      pltpu.VMEM((_RPT,), jnp.float32),
      pltpu.VMEM_SHARED((_N_PAD, dh), jnp.float32),
      pltpu.VMEM_SHARED((_N_PAD, dh), jnp.float32),
      pltpu.VMEM_SHARED((_N_PAD,), jnp.float32),
      pltpu.SemaphoreType.DMA((nslot,)),
      pltpu.SemaphoreType.DMA((nbuf,)),
      pltpu.SemaphoreType.DMA((nbuf,)),
      pltpu.SemaphoreType.DMA((nbuf,)),
  ]

  mesh = plsc.VectorSubcoreMesh(core_axis_name="c", subcore_axis_name="s")

  def body(y_hbm, src_hbm, dst_hbm, *out_and_scratch):
    if with_count:
      out_hbm, cnt_hbm = out_and_scratch[:2]
      sc = out_and_scratch[2:]
    else:
      out_hbm = out_and_scratch[0]
      sc = out_and_scratch[1:]
    src_v, dst_v = sc[0], sc[1]
    rows = sc[2:2 + nbuf]
    (ones_v, zbuf, zcnt, y_s, acc_s, cnt_s,
     isem, gsem, ssem, csem) = sc[2 + nbuf:]

    c = lax.axis_index("c")
    s = lax.axis_index("s")

    pltpu.sync_copy(y_hbm.at[c, pl.ds(s * _RPT, _RPT)],
                    y_s.at[pl.ds(s * _RPT, _RPT)])

    zero = jnp.zeros((_L,), jnp.float32)
    one = jnp.ones((_L,), jnp.float32)
    for i in range(16):
      for j in range(dh // _L):
        zbuf[i, pl.ds(j * _L, _L)] = zero
    for i in range(_RPT // _L):
      zcnt[pl.ds(i * _L, _L)] = zero
    for i in range(_W // _L):
      ones_v[pl.ds(i * _L, _L)] = one

    for i in range(_RPT // 16):
      pltpu.sync_copy(zbuf, acc_s.at[pl.ds(s * _RPT + i * 16, 16)])
    if with_count:
      pltpu.sync_copy(zcnt, cnt_s.at[pl.ds(s * _RPT, _RPT)])

    plsc.subcore_barrier()

    base = s * rows_pt

    def i_start(j):
      sl = j % nslot
      pltpu.async_copy(src_hbm.at[base + j], src_v.at[sl], isem.at[sl])
      pltpu.async_copy(dst_hbm.at[base + j], dst_v.at[sl], isem.at[sl])

    def i_wait(j):
      sl = j % nslot
      pltpu.make_async_copy(src_hbm.at[base + j], src_v.at[sl],
                            isem.at[sl]).wait()
      pltpu.make_async_copy(dst_hbm.at[base + j], dst_v.at[sl],
                            isem.at[sl]).wait()

    def g_start(j, b):
      pltpu.async_copy(y_s.at[src_v.at[j % nslot]], rows[b], gsem.at[b])

    def g_wait(j, b):
      pltpu.make_async_copy(y_s.at[src_v.at[j % nslot]], rows[b],
                            gsem.at[b]).wait()

    def s_start(j, b):
      pltpu.async_copy(rows[b], acc_s.at[dst_v.at[j % nslot]], ssem.at[b],
                       add=True)
      if with_count:
        pltpu.async_copy(ones_v, cnt_s.at[dst_v.at[j % nslot]], csem.at[b],
                         add=True)

    def s_wait(j, b):
      pltpu.make_async_copy(rows[b], acc_s.at[dst_v.at[j % nslot]],
                            ssem.at[b]).wait()
      if with_count:
        pltpu.make_async_copy(ones_v, cnt_s.at[dst_v.at[j % nslot]],
                              csem.at[b]).wait()

    n_it = rows_pt // nbuf
    for j in range(nslot):
      i_start(j)
    for b in range(nbuf):
      i_wait(b)
      g_start(b, b)

    def step(k, carry):
      j0 = k * nbuf
      for b in range(nbuf):
        g_wait(j0 + b, b)
        s_start(j0 + b, b)
      for b in range(nbuf):
        s_wait(j0 + b, b)
        i_wait(j0 + nbuf + b)
        g_start(j0 + nbuf + b, b)
        i_start(j0 + 2 * nbuf + b)
      return carry

    lax.fori_loop(0, n_it - 2, step, 0)

    for k in (n_it - 2, n_it - 1):
      j0 = k * nbuf
      for b in range(nbuf):
        g_wait(j0 + b, b)
        s_start(j0 + b, b)
      for b in range(nbuf):
        s_wait(j0 + b, b)
        if j0 + nbuf + b < rows_pt:
          i_wait(j0 + nbuf + b)
          g_start(j0 + nbuf + b, b)

    plsc.subcore_barrier()

    pltpu.sync_copy(acc_s.at[pl.ds(s * _RPT, _RPT)],
                    out_hbm.at[c, pl.ds(s * _RPT, _RPT)])
    if with_count:
      pltpu.sync_copy(cnt_s.at[pl.ds(s * _RPT, _RPT)],
                      cnt_hbm.at[c, pl.ds(s * _RPT, _RPT)])

  return pl.kernel(body, out_type=tuple(out_type), mesh=mesh,
                   scratch_types=scratch,
                   compiler_params=pltpu.CompilerParams(
                       use_tc_tiling_on_sc=False))


_BLK = 640
_NBLK = _N_PAD // _BLK


def _tc1_body(x_ref, wl_ref, wr_ref, y_ref, r_ref):
  xb = x_ref[...]
  dh = y_ref.shape[2]
  yb = jnp.dot(xb, wl_ref[...], preferred_element_type=jnp.float32)
  y_ref[0] = yb[:, :dh]
  y_ref[1] = yb[:, dh:]
  r_ref[...] = jnp.dot(xb, wr_ref[...], preferred_element_type=jnp.float32)


def _tc2_body(p_ref, cnt_ref, r1_ref, b1_ref, w2l_ref, w2r_ref, y2_ref,
              r2_ref):
  qh = y2_ref.shape[2]
  mean = jnp.concatenate([p_ref[0], p_ref[1]], axis=1) / cnt_ref[...]
  h = jnp.maximum(mean + b1_ref[...] + r1_ref[...], 0.0)
  y2 = jnp.dot(h, w2l_ref[...], preferred_element_type=jnp.float32)
  y2_ref[0] = y2[:, :qh]
  y2_ref[1] = y2[:, qh:]
  r2_ref[...] = jnp.dot(h, w2r_ref[...], preferred_element_type=jnp.float32)


def _tc3_body(p_ref, cnt_ref, r2_ref, b2_ref, wlin_ref, blin_ref,
              out_ref):
  mean = jnp.concatenate([p_ref[0], p_ref[1]], axis=1) / cnt_ref[...]
  g = jnp.maximum(mean + b2_ref[...] + r2_ref[...], 0.0)
  logits = (jnp.dot(g, wlin_ref[...], preferred_element_type=jnp.float32)
            + blin_ref[...])
  m = jnp.max(logits, axis=1, keepdims=True)
  lse = m + jnp.log(jnp.sum(jnp.exp(logits - m), axis=1, keepdims=True))
  out_ref[...] = logits - lse


def _full(shape):
  return pl.BlockSpec(shape, lambda i: tuple(0 for _ in shape))


def kernel(x, edge_index, W1_l, b1, W1_r, W2_l, b2, W2_r, W_lin, b_lin):
  n, d_in = x.shape
  h1 = W1_l.shape[1]
  h2 = W2_l.shape[1]
  ncls = W_lin.shape[1]
  e = edge_index.shape[1]
  dh = h1 // 2

  src = edge_index[0].astype(jnp.int32)
  dst = edge_index[1].astype(jnp.int32)

  e_pt = -(-e // _NS)
  rows_pt = -(-(-(-e_pt // _W)) // 8) * 8
  e_pad = _NS * rows_pt * _W
  pad = e_pad - e
  src2d = jnp.concatenate([src, jnp.zeros((pad,), jnp.int32)]).reshape(-1, _W)
  dst2d = jnp.concatenate(
      [dst, _N + (jnp.arange(pad, dtype=jnp.int32) % (_N_PAD - _N))]
  ).reshape(-1, _W)

  x_pad = jnp.concatenate(
      [x, jnp.zeros((_N_PAD - n, d_in), jnp.float32)])

  y1, r1 = pl.pallas_call(
      _tc1_body,
      grid=(_NBLK,),
      in_specs=[pl.BlockSpec((_BLK, d_in), lambda i: (i, 0)),
                _full((d_in, h1)), _full((d_in, h1))],
      out_specs=[pl.BlockSpec((_NC, _BLK, dh), lambda i: (0, i, 0)),
                 pl.BlockSpec((_BLK, h1), lambda i: (i, 0))],
      out_shape=[jax.ShapeDtypeStruct((_NC, _N_PAD, dh), jnp.float32),
                 jax.ShapeDtypeStruct((_N_PAD, h1), jnp.float32)],
  )(x_pad, W1_l, W1_r)

  p1, c1 = _seg_sum_sc(dh, rows_pt, True)(y1, src2d, dst2d)
  cnt = jnp.maximum(c1[0], 1.0).reshape(_N_PAD, 1)

  qh = h2 // 2
  y2, r2 = pl.pallas_call(
      _tc2_body,
      grid=(_NBLK,),
      in_specs=[pl.BlockSpec((_NC, _BLK, dh), lambda i: (0, i, 0)),
                pl.BlockSpec((_BLK, 1), lambda i: (i, 0)),
                pl.BlockSpec((_BLK, h1), lambda i: (i, 0)),
                _full((1, h1)), _full((h1, h2)), _full((h1, h2))],
      out_specs=[pl.BlockSpec((_NC, _BLK, qh), lambda i: (0, i, 0)),
                 pl.BlockSpec((_BLK, h2), lambda i: (i, 0))],
      out_shape=[jax.ShapeDtypeStruct((_NC, _N_PAD, qh), jnp.float32),
                 jax.ShapeDtypeStruct((_N_PAD, h2), jnp.float32)],
  )(p1, cnt, r1, b1.reshape(1, h1), W2_l, W2_r)

  (p2,) = _seg_sum_sc(qh, rows_pt, False)(y2, src2d, dst2d)

  out = pl.pallas_call(
      _tc3_body,
      grid=(_NBLK,),
      in_specs=[pl.BlockSpec((_NC, _BLK, qh), lambda i: (0, i, 0)),
                pl.BlockSpec((_BLK, 1), lambda i: (i, 0)),
                pl.BlockSpec((_BLK, h2), lambda i: (i, 0)),
                _full((1, h2)),
                _full((h2, ncls)), _full((1, ncls))],
      out_specs=pl.BlockSpec((_BLK, ncls), lambda i: (i, 0)),
      out_shape=jax.ShapeDtypeStruct((_N_PAD, ncls), jnp.float32),
  )(p2, cnt, r2, b2.reshape(1, h2), W_lin, b_lin.reshape(1, ncls))

  return out[:n]
